# baseline (speedup 1.0000x reference)
import functools

import jax
import jax.numpy as jnp
from jax import lax
from jax.experimental import pallas as pl
from jax.experimental.pallas import tpu as pltpu
from jax.experimental.pallas import tpu_sc as plsc

N = 10000
D = 128
NPAD = 10240
CHUNK = 80
NC, NS = 2, 16
NW = NC * NS
RPS = NPAD // NS

_mesh = functools.partial(
    plsc.VectorSubcoreMesh, core_axis_name="c", subcore_axis_name="s",
    num_cores=NC, num_subcores=NS)


CH = 128


def _agg_call(n_rows, d):
  rows_w = n_rows // NW
  assert rows_w * NW == n_rows
  blk = 16 if rows_w % 16 == 0 else 8
  assert rows_w % blk == 0
  nblk = rows_w // blk
  WCH = RPS // CH

  @functools.partial(
      pl.kernel,
      out_type=jax.ShapeDtypeStruct((2, NPAD, d), jnp.float32),
      mesh=_mesh(),
      scratch_types=[
          pltpu.VMEM((blk, CH), jnp.int32),
          pltpu.VMEM((blk, CH), jnp.int32),
          pltpu.VMEM((CH, d), jnp.float32),
          pltpu.VMEM((CH, d), jnp.float32),
          pltpu.VMEM_SHARED((NPAD, d), jnp.float32),
          pltpu.SemaphoreType.DMA,
      ],
  )
  def agg(table, src2d, dst2d, zeros, out, idx_s, idx_d, r0, r1, acc, gsem):
    c = lax.axis_index("c")
    s = lax.axis_index("s")
    w = s * NC + c
    bufs = (r0, r1)
    pltpu.sync_copy(zeros, r0)
    def zloop(i, carry):
      pltpu.sync_copy(r0, acc.at[pl.ds(s * RPS + i * CH, CH)])
      return carry
    lax.fori_loop(0, WCH, zloop, 0)
    plsc.subcore_barrier()

    base_row = w * rows_w
    def body(b, carry):
      pltpu.sync_copy(src2d.at[pl.ds(base_row + b * blk, blk)], idx_s)
      pltpu.sync_copy(dst2d.at[pl.ds(base_row + b * blk, blk)], idx_d)
      for j in range(blk):
        cur = bufs[j % 2]
        pltpu.async_copy(table.at[idx_s.at[j]], cur, gsem).wait()
        pltpu.sync_copy(cur, acc.at[idx_d.at[j]], add=True)
      return carry
    lax.fori_loop(0, nblk, body, 0)
    plsc.subcore_barrier()
    def wloop(i, carry):
      off = s * RPS + i * CH
      pltpu.sync_copy(acc.at[pl.ds(off, CH)], r0)
      pltpu.sync_copy(r0, out.at[c, pl.ds(off, CH)])
      return carry
    lax.fori_loop(0, WCH, wloop, 0)

  return agg


def _count_call(n_edges):
  per_w = n_edges // NW
  assert per_w * NW == n_edges and per_w % CHUNK == 0
  nfull = per_w // CHUNK
  zch = RPS // 8

  @functools.partial(
      pl.kernel,
      out_type=(jax.ShapeDtypeStruct((2 * NPAD,), jnp.float32),
                jax.ShapeDtypeStruct((2 * NPAD,), jnp.float32)),
      mesh=_mesh(),
      scratch_types=[
          pltpu.VMEM((CHUNK,), jnp.int32),
          pltpu.VMEM((CHUNK,), jnp.float32),
          pltpu.VMEM((RPS,), jnp.float32),
          pltpu.VMEM_SHARED((NPAD,), jnp.float32),
          pltpu.VMEM_SHARED((NPAD,), jnp.float32),
      ],
  )
  def count(dst0, dst1, out0, out1, idx, ones_v, cbuf, acc0, acc1):
    c = lax.axis_index("c")
    s = lax.axis_index("s")
    w = s * NC + c
    zero16 = jnp.zeros((16,), jnp.float32)
    for i in range(RPS // 16):
      cbuf[pl.ds(i * 16, 16)] = zero16
    pltpu.sync_copy(cbuf, acc0.at[pl.ds(s * RPS, RPS)])
    pltpu.sync_copy(cbuf, acc1.at[pl.ds(s * RPS, RPS)])
    plsc.subcore_barrier()
    one16 = jnp.ones((16,), jnp.float32)
    for i in range(CHUNK // 16):
      ones_v[pl.ds(i * 16, 16)] = one16
    base_w = w * per_w
    def cloop(dstref, acc):
      def body(j, carry):
        pltpu.sync_copy(dstref.at[pl.ds(base_w + j * CHUNK, CHUNK)], idx)
        pltpu.sync_copy(ones_v, acc.at[idx], add=True)
        return carry
      return body
    lax.fori_loop(0, nfull, cloop(dst0, acc0), 0)
    lax.fori_loop(0, nfull, cloop(dst1, acc1), 0)
    plsc.subcore_barrier()
    for acc, out in ((acc0, out0), (acc1, out1)):
      pltpu.sync_copy(acc.at[pl.ds(s * RPS, RPS)], cbuf)
      pltpu.sync_copy(cbuf, out.at[pl.ds(c * NPAD + s * RPS, RPS)])

  return count


_BN = 2000


def _tc_stage(n_adds, pre=False, scalar_div=False, bias=False, relu=False,
              matmul=False, post=False, bias_after=False, out_d=D):
  def body(*refs):
    refs = list(refs)
    out_ref = refs.pop()
    adds = [refs.pop(0) for _ in range(n_adds)]
    dinv = refs.pop(0)[...] if (pre or post) else None
    dval = refs.pop(0)[0, 0] if scalar_div else None
    b = refs.pop(0)[...] if bias else None
    w = refs.pop(0)[...] if matmul else None
    b2 = refs.pop(0)[...] if bias_after else None
    t = adds[0][...]
    for a in adds[1:]:
      t = t + a[...]
    if scalar_div:
      t = t / dval
    if pre:
      t = t * dinv[:, :1]
    if bias:
      t = t + b
    if relu:
      t = jnp.maximum(t, 0.0)
    if matmul:
      t = jnp.dot(t, w, preferred_element_type=jnp.float32)
    if post:
      t = t * dinv[:, :1]
    if bias_after:
      t = t + b2
    out_ref[...] = t

  specs = [pl.BlockSpec((_BN, D), lambda i: (i, 0)) for _ in range(n_adds)]
  if pre or post:
    specs.append(pl.BlockSpec((_BN, 16), lambda i: (i, 0)))
  if scalar_div:
    specs.append(pl.BlockSpec(memory_space=pltpu.SMEM))
  if bias:
    specs.append(pl.BlockSpec((1, D), lambda i: (0, 0)))
  if matmul:
    specs.append(pl.BlockSpec((D, out_d), lambda i: (0, 0)))
  if bias_after:
    specs.append(pl.BlockSpec((1, out_d), lambda i: (0, 0)))

  return pl.pallas_call(
      body,
      grid=(N // _BN,),
      in_specs=specs,
      out_specs=pl.BlockSpec((_BN, out_d), lambda i: (i, 0)),
      out_shape=jax.ShapeDtypeStruct((N, out_d), jnp.float32),
  )


def _dinv_prep():
  def body(a0, b0, a1, b1, o0, o1):
    o0[...] = jnp.broadcast_to(lax.rsqrt(a0[...] + b0[...] + 1.0), (N, 16))
    o1[...] = jnp.broadcast_to(lax.rsqrt(a1[...] + b1[...] + 1.0), (N, 16))
  return pl.pallas_call(
      body,
      out_shape=(jax.ShapeDtypeStruct((N, 16), jnp.float32),
                 jax.ShapeDtypeStruct((N, 16), jnp.float32)),
  )


def kernel(x0, edge_index0, x1, edge_index1, layer_edge_index1, degrees1,
           W1_0, b1_0, W2_0, b2_0, W1_1, b1_1, W2_1, b2_1,
           Wp1, bp1, Wp2, bp2):
  E = edge_index0.shape[1]
  EC = layer_edge_index1.shape[1]
  src0, dst0 = edge_index0[0], edge_index0[1]
  src1, dst1 = edge_index1[0], edge_index1[1]
  recv, csrc = layer_edge_index1[0], layer_edge_index1[1]

  zeros = jnp.zeros((CH, D), jnp.float32)
  dscal = degrees1[1].reshape(1, 1)

  cnt0, cnt1 = _count_call(E)(dst0, dst1)
  dinv0, dinv1 = _dinv_prep()(cnt0[:N, None], cnt0[NPAD:NPAD + N, None],
                              cnt1[:N, None], cnt1[NPAD:NPAD + N, None])

  def pack(src, dst):
    n_e = src.shape[0]
    rp = -(-n_e // (NW * 4 * CH)) * (NW * 4 * CH)
    p = rp - n_e
    src_p = jnp.concatenate([src, jnp.zeros((p,), src.dtype)])
    dst_p = jnp.concatenate([dst, jnp.full((p,), N, dst.dtype)])
    return src_p.reshape(-1, CH), dst_p.reshape(-1, CH)

  src0p, dst0p = pack(src0, dst0)
  src1p, dst1p = pack(src1, dst1)
  srccp, dstcp = pack(csrc, recv)
  agg_e = _agg_call(src0p.shape[0], D)
  agg_c = _agg_call(srccp.shape[0], D)

  def gcn_net(x, src, dst, dinv, W1, b1, W2, b2):
    hp1 = _tc_stage(1, matmul=True, post=True)(x, dinv, W1)
    s1 = agg_e(hp1, src, dst, zeros)
    hp2 = _tc_stage(3, pre=True, bias=True, relu=True, matmul=True,
                    post=True)(s1[0, :N], s1[1, :N], hp1, dinv,
                               b1.reshape(1, -1), W2)
    s2 = agg_e(hp2, src, dst, zeros)
    return _tc_stage(3, pre=True, bias=True)(s2[0, :N], s2[1, :N], hp2,
                                             dinv, b2.reshape(1, -1))

  last = gcn_net(x0, src0p, dst0p, dinv0, W1_0, b1_0, W2_0, b2_0)
  lf1 = gcn_net(x1, src1p, dst1p, dinv1, W1_1, b1_1, W2_1, b2_1)

  s5 = agg_c(last, srccp, dstcp, zeros)
  hp = _tc_stage(3, scalar_div=True, matmul=True, post=True)(
      lf1, s5[0, :N], s5[1, :N], dinv1, dscal, Wp1)
  s6 = agg_e(hp, src1p, dst1p, zeros)
  u = _tc_stage(3, pre=True, bias=True, relu=True, post=True)(
      s6[0, :N], s6[1, :N], hp, dinv1, bp1.reshape(1, -1))
  s7 = agg_e(u, src1p, dst1p, zeros)
  wp2 = jnp.pad(Wp2, ((0, 0), (0, D - Wp2.shape[1])))
  bp2p = jnp.pad(bp2, (0, D - bp2.shape[0])).reshape(1, D)
  outp = _tc_stage(3, pre=True, matmul=True, bias_after=True)(
      s7[0, :N], s7[1, :N], u, dinv1, wp2, bp2p)
  return outp[:, :Wp2.shape[1]]


# speedup vs baseline: 2.3055x; 2.3055x over previous
import functools

import jax
import jax.numpy as jnp
from jax import lax
from jax.experimental import pallas as pl
from jax.experimental.pallas import tpu as pltpu
from jax.experimental.pallas import tpu_sc as plsc

N = 10000
D = 128
NPAD = 10240
CHUNK = 80
NC, NS = 2, 16
NW = NC * NS
RPS = NPAD // NS

_mesh = functools.partial(
    plsc.VectorSubcoreMesh, core_axis_name="c", subcore_axis_name="s",
    num_cores=NC, num_subcores=NS)


def _agg_call(n_edges, d):
  per_w = n_edges // NW
  assert per_w * NW == n_edges
  ch = max(c for c in (80, 40, 16, 8) if per_w % c == 0)
  nch = per_w // ch
  zc = 40

  @functools.partial(
      pl.kernel,
      out_type=jax.ShapeDtypeStruct((2, NPAD, d), jnp.float32),
      mesh=_mesh(),
      scratch_types=[
          pltpu.VMEM((ch,), jnp.int32),
          pltpu.VMEM((ch,), jnp.int32),
          pltpu.VMEM((ch,), jnp.int32),
          pltpu.VMEM((ch,), jnp.int32),
          pltpu.VMEM((ch, d), jnp.float32),
          pltpu.VMEM((ch, d), jnp.float32),
          pltpu.VMEM_SHARED((NPAD, d), jnp.float32),
          pltpu.SemaphoreType.DMA,
          pltpu.SemaphoreType.DMA,
      ],
  )
  def agg(table, src, dst, zeros, out,
          isa, ida, isb, idb, ra, rb, acc, sema, semb):
    c = lax.axis_index("c")
    s = lax.axis_index("s")
    w = s * NC + c
    pltpu.sync_copy(zeros, ra.at[pl.ds(0, zc)])
    def zloop(i, carry):
      pltpu.sync_copy(ra.at[pl.ds(0, zc)], acc.at[pl.ds(s * RPS + i * zc, zc)])
      return carry
    lax.fori_loop(0, RPS // zc, zloop, 0)
    plsc.subcore_barrier()

    base_w = w * per_w
    def fetch(k, i_s, i_d, rows, sem):
      pltpu.sync_copy(src.at[pl.ds(base_w + k * ch, ch)], i_s)
      pltpu.async_copy(table.at[i_s], rows, sem)
      pltpu.sync_copy(dst.at[pl.ds(base_w + k * ch, ch)], i_d)
    def finish(i_s, i_d, rows, sem):
      pltpu.make_async_copy(table.at[i_s], rows, sem).wait()
      pltpu.sync_copy(rows, acc.at[i_d], add=True)

    fetch(0, isa, ida, ra, sema)
    def body(u, carry):
      k = 2 * u
      fetch(k + 1, isb, idb, rb, semb)
      finish(isa, ida, ra, sema)
      fetch(k + 2, isa, ida, ra, sema)
      finish(isb, idb, rb, semb)
      return carry
    lax.fori_loop(0, (nch - 1) // 2, body, 0)
    finish(isa, ida, ra, sema)
    plsc.subcore_barrier()
    def wloop(i, carry):
      off = s * RPS + i * zc
      pltpu.sync_copy(acc.at[pl.ds(off, zc)], ra.at[pl.ds(0, zc)])
      pltpu.sync_copy(ra.at[pl.ds(0, zc)], out.at[c, pl.ds(off, zc)])
      return carry
    lax.fori_loop(0, RPS // zc, wloop, 0)

  return agg


def _count_call(n_edges):
  per_w = n_edges // NW
  assert per_w * NW == n_edges and per_w % CHUNK == 0
  nfull = per_w // CHUNK
  zch = RPS // 8

  @functools.partial(
      pl.kernel,
      out_type=(jax.ShapeDtypeStruct((2 * NPAD,), jnp.float32),
                jax.ShapeDtypeStruct((2 * NPAD,), jnp.float32)),
      mesh=_mesh(),
      scratch_types=[
          pltpu.VMEM((CHUNK,), jnp.int32),
          pltpu.VMEM((CHUNK,), jnp.float32),
          pltpu.VMEM((RPS,), jnp.float32),
          pltpu.VMEM_SHARED((NPAD,), jnp.float32),
          pltpu.VMEM_SHARED((NPAD,), jnp.float32),
      ],
  )
  def count(dst0, dst1, out0, out1, idx, ones_v, cbuf, acc0, acc1):
    c = lax.axis_index("c")
    s = lax.axis_index("s")
    w = s * NC + c
    zero16 = jnp.zeros((16,), jnp.float32)
    for i in range(RPS // 16):
      cbuf[pl.ds(i * 16, 16)] = zero16
    pltpu.sync_copy(cbuf, acc0.at[pl.ds(s * RPS, RPS)])
    pltpu.sync_copy(cbuf, acc1.at[pl.ds(s * RPS, RPS)])
    plsc.subcore_barrier()
    one16 = jnp.ones((16,), jnp.float32)
    for i in range(CHUNK // 16):
      ones_v[pl.ds(i * 16, 16)] = one16
    base_w = w * per_w
    def cloop(dstref, acc):
      def body(j, carry):
        pltpu.sync_copy(dstref.at[pl.ds(base_w + j * CHUNK, CHUNK)], idx)
        pltpu.sync_copy(ones_v, acc.at[idx], add=True)
        return carry
      return body
    lax.fori_loop(0, nfull, cloop(dst0, acc0), 0)
    lax.fori_loop(0, nfull, cloop(dst1, acc1), 0)
    plsc.subcore_barrier()
    for acc, out in ((acc0, out0), (acc1, out1)):
      pltpu.sync_copy(acc.at[pl.ds(s * RPS, RPS)], cbuf)
      pltpu.sync_copy(cbuf, out.at[pl.ds(c * NPAD + s * RPS, RPS)])

  return count


_BN = 2000


def _tc_stage(n_adds, pre=False, scalar_div=False, bias=False, relu=False,
              matmul=False, post=False, bias_after=False, out_d=D):
  def body(*refs):
    refs = list(refs)
    out_ref = refs.pop()
    adds = [refs.pop(0) for _ in range(n_adds)]
    dinv = refs.pop(0)[...] if (pre or post) else None
    dval = refs.pop(0)[0, 0] if scalar_div else None
    b = refs.pop(0)[...] if bias else None
    w = refs.pop(0)[...] if matmul else None
    b2 = refs.pop(0)[...] if bias_after else None
    t = adds[0][...]
    for a in adds[1:]:
      t = t + a[...]
    if scalar_div:
      t = t / dval
    if pre:
      t = t * dinv[:, :1]
    if bias:
      t = t + b
    if relu:
      t = jnp.maximum(t, 0.0)
    if matmul:
      t = jnp.dot(t, w, preferred_element_type=jnp.float32)
    if post:
      t = t * dinv[:, :1]
    if bias_after:
      t = t + b2
    out_ref[...] = t

  specs = [pl.BlockSpec((_BN, D), lambda i: (i, 0)) for _ in range(n_adds)]
  if pre or post:
    specs.append(pl.BlockSpec((_BN, 16), lambda i: (i, 0)))
  if scalar_div:
    specs.append(pl.BlockSpec(memory_space=pltpu.SMEM))
  if bias:
    specs.append(pl.BlockSpec((1, D), lambda i: (0, 0)))
  if matmul:
    specs.append(pl.BlockSpec((D, out_d), lambda i: (0, 0)))
  if bias_after:
    specs.append(pl.BlockSpec((1, out_d), lambda i: (0, 0)))

  return pl.pallas_call(
      body,
      grid=(N // _BN,),
      in_specs=specs,
      out_specs=pl.BlockSpec((_BN, out_d), lambda i: (i, 0)),
      out_shape=jax.ShapeDtypeStruct((N, out_d), jnp.float32),
  )


def _dinv_prep():
  def body(a0, b0, a1, b1, o0, o1):
    o0[...] = jnp.broadcast_to(lax.rsqrt(a0[...] + b0[...] + 1.0), (N, 16))
    o1[...] = jnp.broadcast_to(lax.rsqrt(a1[...] + b1[...] + 1.0), (N, 16))
  return pl.pallas_call(
      body,
      out_shape=(jax.ShapeDtypeStruct((N, 16), jnp.float32),
                 jax.ShapeDtypeStruct((N, 16), jnp.float32)),
  )


def kernel(x0, edge_index0, x1, edge_index1, layer_edge_index1, degrees1,
           W1_0, b1_0, W2_0, b2_0, W1_1, b1_1, W2_1, b2_1,
           Wp1, bp1, Wp2, bp2):
  E = edge_index0.shape[1]
  EC = layer_edge_index1.shape[1]
  src0, dst0 = edge_index0[0], edge_index0[1]
  src1, dst1 = edge_index1[0], edge_index1[1]
  recv, csrc = layer_edge_index1[0], layer_edge_index1[1]

  zeros = jnp.zeros((40, D), jnp.float32)
  dscal = degrees1[1].reshape(1, 1)

  cnt0, cnt1 = _count_call(E)(dst0, dst1)
  dinv0, dinv1 = _dinv_prep()(cnt0[:N, None], cnt0[NPAD:NPAD + N, None],
                              cnt1[:N, None], cnt1[NPAD:NPAD + N, None])

  agg_e = _agg_call(E, D)
  agg_c = _agg_call(EC, D)

  def gcn_net(x, src, dst, dinv, W1, b1, W2, b2):
    hp1 = _tc_stage(1, matmul=True, post=True)(x, dinv, W1)
    s1 = agg_e(hp1, src, dst, zeros)
    hp2 = _tc_stage(3, pre=True, bias=True, relu=True, matmul=True,
                    post=True)(s1[0, :N], s1[1, :N], hp1, dinv,
                               b1.reshape(1, -1), W2)
    s2 = agg_e(hp2, src, dst, zeros)
    return _tc_stage(3, pre=True, bias=True)(s2[0, :N], s2[1, :N], hp2,
                                             dinv, b2.reshape(1, -1))

  last = gcn_net(x0, src0, dst0, dinv0, W1_0, b1_0, W2_0, b2_0)
  lf1 = gcn_net(x1, src1, dst1, dinv1, W1_1, b1_1, W2_1, b2_1)

  s5 = agg_c(last, csrc, recv, zeros)
  hp = _tc_stage(3, scalar_div=True, matmul=True, post=True)(
      lf1, s5[0, :N], s5[1, :N], dinv1, dscal, Wp1)
  s6 = agg_e(hp, src1, dst1, zeros)
  u = _tc_stage(3, pre=True, bias=True, relu=True, post=True)(
      s6[0, :N], s6[1, :N], hp, dinv1, bp1.reshape(1, -1))
  s7 = agg_e(u, src1, dst1, zeros)
  wp2 = jnp.pad(Wp2, ((0, 0), (0, D - Wp2.shape[1])))
  bp2p = jnp.pad(bp2, (0, D - bp2.shape[0])).reshape(1, D)
  outp = _tc_stage(3, pre=True, matmul=True, bias_after=True)(
      s7[0, :N], s7[1, :N], u, dinv1, wp2, bp2p)
  return outp[:, :Wp2.shape[1]]
